# initial kernel scaffold (unmeasured)
import jax
import jax.numpy as jnp
from jax import lax
from jax.experimental import pallas as pl
from jax.experimental.pallas import tpu as pltpu

N_DEV = 4
M = 4096
K = 1024
N = 8192
CHUNK_M = M // N_DEV
HALF_N = N // 2


def _gemm_body(x_ref, w_ref, out_ref):
    out_ref[...] = jnp.dot(
        x_ref[...].astype(jnp.bfloat16),
        w_ref[...].astype(jnp.bfloat16),
        preferred_element_type=jnp.float32,
    ).astype(jnp.bfloat16)


def _partial_gemm(x, w):
    bm, bn = 512, 2048
    grid = (N // bn, M // bm)
    return pl.pallas_call(
        _gemm_body,
        grid=grid,
        in_specs=[
            pl.BlockSpec((bm, K), lambda n, m: (m, 0)),
            pl.BlockSpec((K, bn), lambda n, m: (0, n)),
        ],
        out_specs=pl.BlockSpec((bm, bn), lambda n, m: (m, n)),
        out_shape=jax.ShapeDtypeStruct((M, N), jnp.bfloat16),
    )(x, w)


def _ar_body(partial_ref, sx_ref, sw_ref, out_ref,
             send_buf, recv_bufs, local_buf, stage,
             send_sems, recv_sems, dma_sem):
    i = lax.axis_index("i")
    left = lax.rem(i - 1 + N_DEV, N_DEV)
    right = lax.rem(i + 1, N_DEV)

    barrier = pltpu.get_barrier_semaphore()
    for nbr in (left, right):
        pl.semaphore_signal(barrier, inc=1, device_id=(nbr,),
                            device_id_type=pl.DeviceIdType.MESH)
    pl.semaphore_wait(barrier, 2)

    scale = sx_ref[0] * sw_ref[0]

    def load_chunk(c, col0, dst):
        cp = pltpu.make_async_copy(
            partial_ref.at[pl.ds(c * CHUNK_M, CHUNK_M), pl.ds(col0, HALF_N)],
            dst, dma_sem)
        cp.start()
        cp.wait()

    def silu(y):
        return y * jax.nn.sigmoid(y)

    for h in range(2):
        col0 = h * HALF_N
        base = h * 6

        for s in range(3):
            c = lax.rem(i - s + N_DEV, N_DEV)
            if s == 0:
                load_chunk(c, col0, send_buf)
            else:
                load_chunk(c, col0, local_buf)
                send_buf[...] = (
                    recv_bufs[(s - 1) % 2].astype(jnp.float32)
                    + local_buf[...].astype(jnp.float32)
                ).astype(jnp.bfloat16)
            rdma = pltpu.make_async_remote_copy(
                src_ref=send_buf,
                dst_ref=recv_bufs.at[s % 2],
                send_sem=send_sems.at[base + s],
                recv_sem=recv_sems.at[base + s],
                device_id=(right,),
                device_id_type=pl.DeviceIdType.MESH,
            )
            rdma.start()
            rdma.wait()

        o = lax.rem(i + 1, N_DEV)
        load_chunk(o, col0, local_buf)
        y = (recv_bufs[0].astype(jnp.float32)
             + local_buf[...].astype(jnp.float32)) * scale
        stage[...] = silu(y)
        cp = pltpu.make_async_copy(
            stage,
            out_ref.at[pl.ds(o * CHUNK_M, CHUNK_M), pl.ds(col0, HALF_N)],
            dma_sem)
        cp.start()
        cp.wait()
        send_buf[...] = stage[...].astype(jnp.bfloat16)

        for t in range(3):
            src = send_buf if t == 0 else recv_bufs.at[t % 2]
            rdma = pltpu.make_async_remote_copy(
                src_ref=src,
                dst_ref=recv_bufs.at[(t + 1) % 2],
                send_sem=send_sems.at[base + 3 + t],
                recv_sem=recv_sems.at[base + 3 + t],
                device_id=(right,),
                device_id_type=pl.DeviceIdType.MESH,
            )
            rdma.start()
            rdma.wait()
            g = lax.rem(i - t + N_DEV, N_DEV)
            stage[...] = recv_bufs[(t + 1) % 2].astype(jnp.float32)
            cp = pltpu.make_async_copy(
                stage,
                out_ref.at[pl.ds(g * CHUNK_M, CHUNK_M), pl.ds(col0, HALF_N)],
                dma_sem)
            cp.start()
            cp.wait()


def _allreduce_silu(partial, scale_x, scale_w):
    return pl.pallas_call(
        _ar_body,
        in_specs=[
            pl.BlockSpec(memory_space=pltpu.ANY),
            pl.BlockSpec(memory_space=pltpu.SMEM),
            pl.BlockSpec(memory_space=pltpu.SMEM),
        ],
        out_specs=pl.BlockSpec(memory_space=pltpu.ANY),
        out_shape=jax.ShapeDtypeStruct((M, N), jnp.float32),
        scratch_shapes=[
            pltpu.VMEM((CHUNK_M, HALF_N), jnp.bfloat16),
            pltpu.VMEM((2, CHUNK_M, HALF_N), jnp.bfloat16),
            pltpu.VMEM((CHUNK_M, HALF_N), jnp.bfloat16),
            pltpu.VMEM((CHUNK_M, HALF_N), jnp.float32),
            pltpu.SemaphoreType.DMA((12,)),
            pltpu.SemaphoreType.DMA((12,)),
            pltpu.SemaphoreType.DMA,
        ],
        compiler_params=pltpu.CompilerParams(collective_id=0),
    )(partial, scale_x, scale_w)


def kernel(x, w_mat, scale_x, scale_w):
    partial = _partial_gemm(x, w_mat)
    return _allreduce_silu(partial, scale_x, scale_w)


# baseline (device time: 1383036 ns/iter reference)
import jax
import jax.numpy as jnp
from jax import lax
from jax.experimental import pallas as pl
from jax.experimental.pallas import tpu as pltpu

N_DEV = 4
M = 4096
K = 1024
N = 8192
CHUNK_M = M // N_DEV
HALF_N = N // 2


def _gemm_body(x_ref, w_ref, out_ref):
    out_ref[...] = jnp.dot(
        x_ref[...].astype(jnp.bfloat16),
        w_ref[...].astype(jnp.bfloat16),
        preferred_element_type=jnp.float32,
    ).astype(jnp.bfloat16)


def _partial_gemm(x, w):
    bm, bn = 512, 2048
    grid = (N // bn, M // bm)
    return pl.pallas_call(
        _gemm_body,
        grid=grid,
        in_specs=[
            pl.BlockSpec((bm, K), lambda n, m: (m, 0)),
            pl.BlockSpec((K, bn), lambda n, m: (0, n)),
        ],
        out_specs=pl.BlockSpec((bm, bn), lambda n, m: (m, n)),
        out_shape=jax.ShapeDtypeStruct((M, N), jnp.bfloat16),
    )(x, w)


def _ar_body(partial_ref, sx_ref, sw_ref, out_ref,
             send_buf, recv_bufs, local_buf, stage,
             send_sems, recv_sems, dma_sem):
    i = lax.axis_index("i")
    left = lax.rem(i - 1 + N_DEV, N_DEV)
    right = lax.rem(i + 1, N_DEV)

    barrier = pltpu.get_barrier_semaphore()
    for nbr in (left, right):
        pl.semaphore_signal(barrier, inc=1, device_id=(nbr,),
                            device_id_type=pl.DeviceIdType.MESH)
    pl.semaphore_wait(barrier, 2)

    scale = sx_ref[0] * sw_ref[0]

    def load_chunk(c, col0, dst):
        cp = pltpu.make_async_copy(
            partial_ref.at[pl.ds(c * CHUNK_M, CHUNK_M), pl.ds(col0, HALF_N)],
            dst, dma_sem)
        cp.start()
        cp.wait()

    def silu(y):
        return y * jax.nn.sigmoid(y)

    for h in range(2):
        col0 = h * HALF_N
        base = h * 6

        for s in range(3):
            c = lax.rem(i - s + N_DEV, N_DEV)
            if s == 0:
                load_chunk(c, col0, send_buf)
            else:
                load_chunk(c, col0, local_buf)
                send_buf[...] = (
                    recv_bufs[(s - 1) % 2].astype(jnp.float32)
                    + local_buf[...].astype(jnp.float32)
                ).astype(jnp.bfloat16)
            rdma = pltpu.make_async_remote_copy(
                src_ref=send_buf,
                dst_ref=recv_bufs.at[s % 2],
                send_sem=send_sems.at[base + s],
                recv_sem=recv_sems.at[base + s],
                device_id=(right,),
                device_id_type=pl.DeviceIdType.MESH,
            )
            rdma.start()
            rdma.wait()

        o = lax.rem(i + 1, N_DEV)
        load_chunk(o, col0, local_buf)
        y = (recv_bufs[0].astype(jnp.float32)
             + local_buf[...].astype(jnp.float32)) * scale
        stage[...] = silu(y)
        cp = pltpu.make_async_copy(
            stage,
            out_ref.at[pl.ds(o * CHUNK_M, CHUNK_M), pl.ds(col0, HALF_N)],
            dma_sem)
        cp.start()
        cp.wait()
        send_buf[...] = stage[...].astype(jnp.bfloat16)

        for t in range(3):
            src = send_buf if t == 0 else recv_bufs.at[t % 2]
            rdma = pltpu.make_async_remote_copy(
                src_ref=src,
                dst_ref=recv_bufs.at[(t + 1) % 2],
                send_sem=send_sems.at[base + 3 + t],
                recv_sem=recv_sems.at[base + 3 + t],
                device_id=(right,),
                device_id_type=pl.DeviceIdType.MESH,
            )
            rdma.start()
            rdma.wait()
            g = lax.rem(i - t + N_DEV, N_DEV)
            stage[...] = recv_bufs[(t + 1) % 2].astype(jnp.float32)
            cp = pltpu.make_async_copy(
                stage,
                out_ref.at[pl.ds(g * CHUNK_M, CHUNK_M), pl.ds(col0, HALF_N)],
                dma_sem)
            cp.start()
            cp.wait()


def _allreduce_silu(partial, scale_x, scale_w):
    return pl.pallas_call(
        _ar_body,
        in_specs=[
            pl.BlockSpec(memory_space=pl.ANY),
            pl.BlockSpec(memory_space=pltpu.SMEM),
            pl.BlockSpec(memory_space=pltpu.SMEM),
        ],
        out_specs=pl.BlockSpec(memory_space=pl.ANY),
        out_shape=jax.ShapeDtypeStruct((M, N), jnp.float32),
        scratch_shapes=[
            pltpu.VMEM((CHUNK_M, HALF_N), jnp.bfloat16),
            pltpu.VMEM((2, CHUNK_M, HALF_N), jnp.bfloat16),
            pltpu.VMEM((CHUNK_M, HALF_N), jnp.bfloat16),
            pltpu.VMEM((CHUNK_M, HALF_N), jnp.float32),
            pltpu.SemaphoreType.DMA((12,)),
            pltpu.SemaphoreType.DMA((12,)),
            pltpu.SemaphoreType.DMA,
        ],
        compiler_params=pltpu.CompilerParams(
            collective_id=0, vmem_limit_bytes=100 * 1024 * 1024),
    )(partial, scale_x, scale_w)


def kernel(x, w_mat, scale_x, scale_w):
    partial = _partial_gemm(x, w_mat)
    return _allreduce_silu(partial, scale_x, scale_w)


# device time: 829969 ns/iter; 1.6664x vs baseline; 1.6664x over previous
import jax
import jax.numpy as jnp
from jax import lax
from jax.experimental import pallas as pl
from jax.experimental.pallas import tpu as pltpu

N_DEV = 4
M = 4096
K = 1024
N = 8192
CHUNK_M = M // N_DEV
CHUNK_COLS = N // 4
HALF_N = N // 2


def _gemm_body(x_ref, w_ref, out_ref):
    out_ref[...] = jnp.dot(
        x_ref[...].astype(jnp.bfloat16),
        w_ref[...].astype(jnp.bfloat16),
        preferred_element_type=jnp.float32,
    ).astype(jnp.bfloat16)


def _partial_gemm(x, w):
    bm, bn = 512, 2048
    grid = (N // bn, M // bm)
    return pl.pallas_call(
        _gemm_body,
        grid=grid,
        in_specs=[
            pl.BlockSpec((bm, K), lambda n, m: (m, 0)),
            pl.BlockSpec((K, bn), lambda n, m: (0, n)),
        ],
        out_specs=pl.BlockSpec((bm, bn), lambda n, m: (m, n)),
        out_shape=jax.ShapeDtypeStruct((M, N), jnp.bfloat16),
    )(x, w)


def _ar_body(partial_ref, sx_ref, sw_ref, out_ref,
             send_bufs, recv_bufs, local_bufs, stages,
             send_sems, recv_sems, dma_sems):
    i = lax.axis_index("i")
    left = lax.rem(i - 1 + N_DEV, N_DEV)
    right = lax.rem(i + 1, N_DEV)

    barrier = pltpu.get_barrier_semaphore()
    for nbr in (left, right):
        pl.semaphore_signal(barrier, inc=1, device_id=(nbr,),
                            device_id_type=pl.DeviceIdType.MESH)
    pl.semaphore_wait(barrier, 2)

    scale = sx_ref[0] * sw_ref[0]

    def start_load(c, col0, dst, d):
        cp = pltpu.make_async_copy(
            partial_ref.at[pl.ds(c * CHUNK_M, CHUNK_M),
                           pl.ds(col0, CHUNK_COLS)],
            dst, dma_sems.at[d])
        cp.start()
        return cp

    def start_store(src, g, col0, d):
        cp = pltpu.make_async_copy(
            src,
            out_ref.at[pl.ds(g * CHUNK_M, CHUNK_M),
                       pl.ds(col0, CHUNK_COLS)],
            dma_sems.at[d])
        cp.start()
        return cp

    def silu(y):
        return y * jax.nn.sigmoid(y)

    for p in range(2):
        col0s = (p * CHUNK_COLS, HALF_N + p * CHUNK_COLS)
        targets = (right, left)
        base = p * 6

        def rs_chunk(s, d):
            return lax.rem(i + (s if d else -s) + N_DEV, N_DEV)

        for s in range(3):
            if s == 0:
                cps = [start_load(rs_chunk(0, d), col0s[d],
                                  send_bufs.at[d], d) for d in range(2)]
                for cp in cps:
                    cp.wait()
            else:
                cps = [start_load(rs_chunk(s, d), col0s[d],
                                  local_bufs.at[d], d) for d in range(2)]
                for d in range(2):
                    cps[d].wait()
                    send_bufs[d] = (
                        recv_bufs[d, (s - 1) % 2].astype(jnp.float32)
                        + local_bufs[d].astype(jnp.float32)
                    ).astype(jnp.bfloat16)
            rdmas = []
            for d in range(2):
                rdma = pltpu.make_async_remote_copy(
                    src_ref=send_bufs.at[d],
                    dst_ref=recv_bufs.at[d, s % 2],
                    send_sem=send_sems.at[d * 12 + base + s],
                    recv_sem=recv_sems.at[d * 12 + base + s],
                    device_id=(targets[d],),
                    device_id_type=pl.DeviceIdType.MESH,
                )
                rdma.start()
                rdmas.append(rdma)
            for rdma in rdmas:
                rdma.wait()

        owned = (lax.rem(i + 1, N_DEV), lax.rem(i - 1 + N_DEV, N_DEV))
        cps = [start_load(owned[d], col0s[d], local_bufs.at[d], d)
               for d in range(2)]
        store_cps = []
        for d in range(2):
            cps[d].wait()
            y = (recv_bufs[d, 0].astype(jnp.float32)
                 + local_bufs[d].astype(jnp.float32)) * scale
            stages[d] = silu(y)
            store_cps.append(start_store(stages.at[d], owned[d],
                                         col0s[d], d))
            send_bufs[d] = stages[d].astype(jnp.bfloat16)
        for cp in store_cps:
            cp.wait()

        for t in range(3):
            rdmas = []
            for d in range(2):
                src = send_bufs.at[d] if t == 0 else recv_bufs.at[d, t % 2]
                rdma = pltpu.make_async_remote_copy(
                    src_ref=src,
                    dst_ref=recv_bufs.at[d, (t + 1) % 2],
                    send_sem=send_sems.at[d * 12 + base + 3 + t],
                    recv_sem=recv_sems.at[d * 12 + base + 3 + t],
                    device_id=(targets[d],),
                    device_id_type=pl.DeviceIdType.MESH,
                )
                rdma.start()
                rdmas.append(rdma)
            store_cps = []
            for d in range(2):
                rdmas[d].wait()
                g = lax.rem(i + (t if d else -t) + N_DEV, N_DEV)
                stages[d] = recv_bufs[d, (t + 1) % 2].astype(jnp.float32)
                store_cps.append(start_store(stages.at[d], g, col0s[d], d))
            for cp in store_cps:
                cp.wait()


def _allreduce_silu(partial, scale_x, scale_w):
    return pl.pallas_call(
        _ar_body,
        in_specs=[
            pl.BlockSpec(memory_space=pl.ANY),
            pl.BlockSpec(memory_space=pltpu.SMEM),
            pl.BlockSpec(memory_space=pltpu.SMEM),
        ],
        out_specs=pl.BlockSpec(memory_space=pl.ANY),
        out_shape=jax.ShapeDtypeStruct((M, N), jnp.float32),
        scratch_shapes=[
            pltpu.VMEM((2, CHUNK_M, CHUNK_COLS), jnp.bfloat16),
            pltpu.VMEM((2, 2, CHUNK_M, CHUNK_COLS), jnp.bfloat16),
            pltpu.VMEM((2, CHUNK_M, CHUNK_COLS), jnp.bfloat16),
            pltpu.VMEM((2, CHUNK_M, CHUNK_COLS), jnp.float32),
            pltpu.SemaphoreType.DMA((24,)),
            pltpu.SemaphoreType.DMA((24,)),
            pltpu.SemaphoreType.DMA((2,)),
        ],
        compiler_params=pltpu.CompilerParams(
            collective_id=0, vmem_limit_bytes=100 * 1024 * 1024),
    )(partial, scale_x, scale_w)


def kernel(x, w_mat, scale_x, scale_w):
    partial = _partial_gemm(x, w_mat)
    return _allreduce_silu(partial, scale_x, scale_w)


# device time: 788105 ns/iter; 1.7549x vs baseline; 1.0531x over previous
import jax
import jax.numpy as jnp
from jax import lax
from jax.experimental import pallas as pl
from jax.experimental.pallas import tpu as pltpu

N_DEV = 4
M = 4096
K = 1024
N = 8192
CHUNK_M = M // N_DEV
CHUNK_COLS = N // 4
HALF_N = N // 2


def _ar_body(x_ref, w_ref, sx_ref, sw_ref, out_ref,
             send_bufs, recv_bufs, w_bufs, stage,
             send_sems, recv_sems, dma_sems):
    i = lax.axis_index("i")
    left = lax.rem(i - 1 + N_DEV, N_DEV)
    right = lax.rem(i + 1, N_DEV)

    barrier = pltpu.get_barrier_semaphore()
    for nbr in (left, right):
        pl.semaphore_signal(barrier, inc=1, device_id=(nbr,),
                            device_id_type=pl.DeviceIdType.MESH)
    pl.semaphore_wait(barrier, 2)

    scale = sx_ref[0] * sw_ref[0]

    def dot_chunk(c, d):
        return jnp.dot(x_ref[pl.ds(c * CHUNK_M, CHUNK_M), :], w_bufs[d],
                       preferred_element_type=jnp.float32)

    pending = [None]

    def do_store(value, g, col0):
        if pending[0] is not None:
            pending[0].wait()
        stage[...] = value
        cp = pltpu.make_async_copy(
            stage,
            out_ref.at[pl.ds(g * CHUNK_M, CHUNK_M),
                       pl.ds(col0, CHUNK_COLS)],
            dma_sems.at[0])
        cp.start()
        pending[0] = cp

    for p in range(2):
        col0s = (p * CHUNK_COLS, HALF_N + p * CHUNK_COLS)
        targets = (right, left)
        base = p * 6

        def ring_chunk(t, d):
            return lax.rem(i + (t if d else -t) + N_DEV, N_DEV)

        if pending[0] is not None:
            pending[0].wait()
            pending[0] = None
        wcps = []
        for d in range(2):
            cp = pltpu.make_async_copy(
                w_ref.at[:, pl.ds(col0s[d], CHUNK_COLS)],
                w_bufs.at[d], dma_sems.at[d])
            cp.start()
            wcps.append(cp)
        for cp in wcps:
            cp.wait()

        for d in range(2):
            send_bufs[d] = dot_chunk(ring_chunk(0, d), d).astype(jnp.bfloat16)

        for s in range(3):
            rdmas = []
            for d in range(2):
                rdma = pltpu.make_async_remote_copy(
                    src_ref=send_bufs.at[d],
                    dst_ref=recv_bufs.at[d, s % 2],
                    send_sem=send_sems.at[d * 12 + base + s],
                    recv_sem=recv_sems.at[d * 12 + base + s],
                    device_id=(targets[d],),
                    device_id_type=pl.DeviceIdType.MESH,
                )
                rdma.start()
                rdmas.append(rdma)
            for rdma in rdmas:
                rdma.wait()
            if s < 2:
                for d in range(2):
                    send_bufs[d] = (
                        recv_bufs[d, s % 2].astype(jnp.float32)
                        + dot_chunk(ring_chunk(s + 1, d), d)
                    ).astype(jnp.bfloat16)
            else:
                for d in range(2):
                    y = (recv_bufs[d, 0].astype(jnp.float32)
                         + dot_chunk(ring_chunk(3, d), d)) * scale
                    silu = y * jax.nn.sigmoid(y)
                    send_bufs[d] = silu.astype(jnp.bfloat16)
                    do_store(silu, ring_chunk(3, d), col0s[d])

        for t in range(3):
            rdmas = []
            for d in range(2):
                src = send_bufs.at[d] if t == 0 else recv_bufs.at[d, t % 2]
                rdma = pltpu.make_async_remote_copy(
                    src_ref=src,
                    dst_ref=recv_bufs.at[d, (t + 1) % 2],
                    send_sem=send_sems.at[d * 12 + base + 3 + t],
                    recv_sem=recv_sems.at[d * 12 + base + 3 + t],
                    device_id=(targets[d],),
                    device_id_type=pl.DeviceIdType.MESH,
                )
                rdma.start()
                rdmas.append(rdma)
            if t > 0:
                for d in range(2):
                    do_store(recv_bufs[d, t % 2].astype(jnp.float32),
                             ring_chunk(t - 1, d), col0s[d])
            for rdma in rdmas:
                rdma.wait()
        for d in range(2):
            do_store(recv_bufs[d, 1].astype(jnp.float32),
                     ring_chunk(2, d), col0s[d])
    pending[0].wait()


def kernel(x, w_mat, scale_x, scale_w):
    x_bf = x.astype(jnp.bfloat16)
    w_bf = w_mat.astype(jnp.bfloat16)
    return pl.pallas_call(
        _ar_body,
        in_specs=[
            pl.BlockSpec(memory_space=pltpu.VMEM),
            pl.BlockSpec(memory_space=pl.ANY),
            pl.BlockSpec(memory_space=pltpu.SMEM),
            pl.BlockSpec(memory_space=pltpu.SMEM),
        ],
        out_specs=pl.BlockSpec(memory_space=pl.ANY),
        out_shape=jax.ShapeDtypeStruct((M, N), jnp.float32),
        scratch_shapes=[
            pltpu.VMEM((2, CHUNK_M, CHUNK_COLS), jnp.bfloat16),
            pltpu.VMEM((2, 2, CHUNK_M, CHUNK_COLS), jnp.bfloat16),
            pltpu.VMEM((2, K, CHUNK_COLS), jnp.bfloat16),
            pltpu.VMEM((CHUNK_M, CHUNK_COLS), jnp.float32),
            pltpu.SemaphoreType.DMA((24,)),
            pltpu.SemaphoreType.DMA((24,)),
            pltpu.SemaphoreType.DMA((2,)),
        ],
        compiler_params=pltpu.CompilerParams(
            collective_id=0, vmem_limit_bytes=62 * 1024 * 1024),
    )(x_bf, w_bf, scale_x, scale_w)


# device time: 749324 ns/iter; 1.8457x vs baseline; 1.0518x over previous
import jax
import jax.numpy as jnp
from jax import lax
from jax.experimental import pallas as pl
from jax.experimental.pallas import tpu as pltpu

N_DEV = 4
M = 4096
K = 1024
N = 8192
CHUNK_M = M // N_DEV
CHUNK_COLS = N // 4
HALF_N = N // 2


def _ar_body(x_ref, w_ref, sx_ref, sw_ref, out_ref,
             send_bufs, recv_bufs, local_bufs, w_bufs, stage,
             send_sems, recv_sems, dma_sems):
    i = lax.axis_index("i")
    left = lax.rem(i - 1 + N_DEV, N_DEV)
    right = lax.rem(i + 1, N_DEV)

    barrier = pltpu.get_barrier_semaphore()
    for nbr in (left, right):
        pl.semaphore_signal(barrier, inc=1, device_id=(nbr,),
                            device_id_type=pl.DeviceIdType.MESH)
    pl.semaphore_wait(barrier, 2)

    scale = sx_ref[0] * sw_ref[0]

    def dot_chunk(c, d):
        return jnp.dot(x_ref[pl.ds(c * CHUNK_M, CHUNK_M), :], w_bufs[d],
                       preferred_element_type=jnp.float32)

    pending = [None]

    def do_store(value, g, col0):
        if pending[0] is not None:
            pending[0].wait()
        stage[...] = value
        cp = pltpu.make_async_copy(
            stage,
            out_ref.at[pl.ds(g * CHUNK_M, CHUNK_M),
                       pl.ds(col0, CHUNK_COLS)],
            dma_sems.at[0])
        cp.start()
        pending[0] = cp

    for p in range(2):
        col0s = (p * CHUNK_COLS, HALF_N + p * CHUNK_COLS)
        targets = (right, left)
        base = p * 6

        def ring_chunk(t, d):
            return lax.rem(i + (t if d else -t) + N_DEV, N_DEV)

        if pending[0] is not None:
            pending[0].wait()
            pending[0] = None
        wcps = []
        for d in range(2):
            cp = pltpu.make_async_copy(
                w_ref.at[:, pl.ds(col0s[d], CHUNK_COLS)],
                w_bufs.at[d], dma_sems.at[d])
            cp.start()
            wcps.append(cp)
        for cp in wcps:
            cp.wait()

        for d in range(2):
            send_bufs[d] = dot_chunk(ring_chunk(0, d), d).astype(jnp.bfloat16)

        for s in range(3):
            rdmas = []
            for d in range(2):
                rdma = pltpu.make_async_remote_copy(
                    src_ref=send_bufs.at[d],
                    dst_ref=recv_bufs.at[d, s % 2],
                    send_sem=send_sems.at[d * 12 + base + s],
                    recv_sem=recv_sems.at[d * 12 + base + s],
                    device_id=(targets[d],),
                    device_id_type=pl.DeviceIdType.MESH,
                )
                rdma.start()
                rdmas.append(rdma)
            for d in range(2):
                local_bufs[d] = dot_chunk(ring_chunk(s + 1, d),
                                          d).astype(jnp.bfloat16)
            for rdma in rdmas:
                rdma.wait()
            if s < 2:
                for d in range(2):
                    send_bufs[d] = (
                        recv_bufs[d, s % 2].astype(jnp.float32)
                        + local_bufs[d].astype(jnp.float32)
                    ).astype(jnp.bfloat16)
            else:
                for d in range(2):
                    y = (recv_bufs[d, 0].astype(jnp.float32)
                         + local_bufs[d].astype(jnp.float32)) * scale
                    silu = y * jax.nn.sigmoid(y)
                    send_bufs[d] = silu.astype(jnp.bfloat16)
                    do_store(silu, ring_chunk(3, d), col0s[d])

        for t in range(3):
            rdmas = []
            for d in range(2):
                src = send_bufs.at[d] if t == 0 else recv_bufs.at[d, t % 2]
                rdma = pltpu.make_async_remote_copy(
                    src_ref=src,
                    dst_ref=recv_bufs.at[d, (t + 1) % 2],
                    send_sem=send_sems.at[d * 12 + base + 3 + t],
                    recv_sem=recv_sems.at[d * 12 + base + 3 + t],
                    device_id=(targets[d],),
                    device_id_type=pl.DeviceIdType.MESH,
                )
                rdma.start()
                rdmas.append(rdma)
            if t > 0:
                for d in range(2):
                    do_store(recv_bufs[d, t % 2].astype(jnp.float32),
                             ring_chunk(t - 1, d), col0s[d])
            for rdma in rdmas:
                rdma.wait()
        for d in range(2):
            do_store(recv_bufs[d, 1].astype(jnp.float32),
                     ring_chunk(2, d), col0s[d])
    pending[0].wait()


def kernel(x, w_mat, scale_x, scale_w):
    x_bf = x.astype(jnp.bfloat16)
    w_bf = w_mat.astype(jnp.bfloat16)
    return pl.pallas_call(
        _ar_body,
        in_specs=[
            pl.BlockSpec(memory_space=pltpu.VMEM),
            pl.BlockSpec(memory_space=pl.ANY),
            pl.BlockSpec(memory_space=pltpu.SMEM),
            pl.BlockSpec(memory_space=pltpu.SMEM),
        ],
        out_specs=pl.BlockSpec(memory_space=pl.ANY),
        out_shape=jax.ShapeDtypeStruct((M, N), jnp.float32),
        scratch_shapes=[
            pltpu.VMEM((2, CHUNK_M, CHUNK_COLS), jnp.bfloat16),
            pltpu.VMEM((2, 2, CHUNK_M, CHUNK_COLS), jnp.bfloat16),
            pltpu.VMEM((2, CHUNK_M, CHUNK_COLS), jnp.bfloat16),
            pltpu.VMEM((2, K, CHUNK_COLS), jnp.bfloat16),
            pltpu.VMEM((CHUNK_M, CHUNK_COLS), jnp.float32),
            pltpu.SemaphoreType.DMA((24,)),
            pltpu.SemaphoreType.DMA((24,)),
            pltpu.SemaphoreType.DMA((2,)),
        ],
        compiler_params=pltpu.CompilerParams(
            collective_id=0, vmem_limit_bytes=62 * 1024 * 1024),
    )(x_bf, w_bf, scale_x, scale_w)


# device time: 700421 ns/iter; 1.9746x vs baseline; 1.0698x over previous
import jax
import jax.numpy as jnp
from jax import lax
from jax.experimental import pallas as pl
from jax.experimental.pallas import tpu as pltpu

N_DEV = 4
M = 4096
K = 1024
N = 8192
CHUNK_M = M // N_DEV
CHUNK_COLS = N // 4
HALF_N = N // 2


def _ar_body(x_ref, w_ref, sx_ref, sw_ref, out_ref,
             send_bufs, recv_bufs, local_bufs, w_bufs,
             send_sems, recv_sems, dma_sems):
    i = lax.axis_index("i")
    left = lax.rem(i - 1 + N_DEV, N_DEV)
    right = lax.rem(i + 1, N_DEV)

    barrier = pltpu.get_barrier_semaphore()
    for nbr in (left, right):
        pl.semaphore_signal(barrier, inc=1, device_id=(nbr,),
                            device_id_type=pl.DeviceIdType.MESH)
    pl.semaphore_wait(barrier, 2)

    scale = sx_ref[0] * sw_ref[0]

    def dot_chunk(c, d):
        return jnp.dot(x_ref[pl.ds(c * CHUNK_M, CHUNK_M), :], w_bufs[d],
                       preferred_element_type=jnp.float32)

    def start_store(src, g, col0, d):
        cp = pltpu.make_async_copy(
            src,
            out_ref.at[pl.ds(g * CHUNK_M, CHUNK_M),
                       pl.ds(col0, CHUNK_COLS)],
            dma_sems.at[d])
        cp.start()
        return cp

    def start_w_load(col0, d):
        cp = pltpu.make_async_copy(
            w_ref.at[:, pl.ds(col0, CHUNK_COLS)],
            w_bufs.at[d], dma_sems.at[2 + d])
        cp.start()
        return cp

    def all_col0s(p):
        return (p * CHUNK_COLS, HALF_N + p * CHUNK_COLS)

    targets = (right, left)
    pending = [None, None]

    for p in range(2):
        col0s = all_col0s(p)
        base = p * 6

        def ring_chunk(t, d):
            return lax.rem(i + (t if d else -t) + N_DEV, N_DEV)

        if p == 0:
            wcps = [start_w_load(col0s[d], d) for d in range(2)]
            for cp in wcps:
                cp.wait()
            for d in range(2):
                send_bufs[d] = dot_chunk(ring_chunk(0, d),
                                         d).astype(jnp.bfloat16)

        for s in range(3):
            rdmas = []
            for d in range(2):
                rdma = pltpu.make_async_remote_copy(
                    src_ref=send_bufs.at[d],
                    dst_ref=recv_bufs.at[d, s % 2],
                    send_sem=send_sems.at[d * 12 + base + s],
                    recv_sem=recv_sems.at[d * 12 + base + s],
                    device_id=(targets[d],),
                    device_id_type=pl.DeviceIdType.MESH,
                )
                rdma.start()
                rdmas.append(rdma)
            for d in range(2):
                local_bufs[d] = dot_chunk(ring_chunk(s + 1, d),
                                          d).astype(jnp.bfloat16)
            for rdma in rdmas:
                rdma.wait()
            if s < 2:
                for d in range(2):
                    send_bufs[d] = (
                        recv_bufs[d, s % 2].astype(jnp.float32)
                        + local_bufs[d].astype(jnp.float32)
                    ).astype(jnp.bfloat16)
            else:
                for d in range(2):
                    y = (recv_bufs[d, 0].astype(jnp.float32)
                         + local_bufs[d].astype(jnp.float32)) * scale
                    silu = y * jax.nn.sigmoid(y)
                    send_bufs[d] = silu.astype(jnp.bfloat16)
                    if pending[d] is not None:
                        pending[d].wait()
                    pending[d] = start_store(send_bufs.at[d],
                                             ring_chunk(3, d), col0s[d], d)

        for t in range(3):
            rdmas = []
            for d in range(2):
                src = send_bufs.at[d] if t == 0 else recv_bufs.at[d, t % 2]
                rdma = pltpu.make_async_remote_copy(
                    src_ref=src,
                    dst_ref=recv_bufs.at[d, (t + 1) % 2],
                    send_sem=send_sems.at[d * 12 + base + 3 + t],
                    recv_sem=recv_sems.at[d * 12 + base + 3 + t],
                    device_id=(targets[d],),
                    device_id_type=pl.DeviceIdType.MESH,
                )
                rdma.start()
                rdmas.append(rdma)
            if p == 0 and t == 0:
                wnext = [start_w_load(all_col0s(1)[d], d) for d in range(2)]
            if t > 0:
                for d in range(2):
                    pending[d].wait()
                    pending[d] = start_store(recv_bufs.at[d, t % 2],
                                             ring_chunk(t - 1, d),
                                             col0s[d], d)
            if p == 0 and t == 1:
                for cp in wnext:
                    cp.wait()
                for d in range(2):
                    send_bufs[d] = dot_chunk(ring_chunk(0, d),
                                             d).astype(jnp.bfloat16)
            for rdma in rdmas:
                rdma.wait()
        for d in range(2):
            pending[d].wait()
            pending[d] = start_store(recv_bufs.at[d, 1],
                                     ring_chunk(2, d), col0s[d], d)
    for d in range(2):
        pending[d].wait()


def kernel(x, w_mat, scale_x, scale_w):
    x_bf = x.astype(jnp.bfloat16)
    w_bf = w_mat.astype(jnp.bfloat16)
    out = pl.pallas_call(
        _ar_body,
        in_specs=[
            pl.BlockSpec(memory_space=pltpu.VMEM),
            pl.BlockSpec(memory_space=pl.ANY),
            pl.BlockSpec(memory_space=pltpu.SMEM),
            pl.BlockSpec(memory_space=pltpu.SMEM),
        ],
        out_specs=pl.BlockSpec(memory_space=pl.ANY),
        out_shape=jax.ShapeDtypeStruct((M, N), jnp.bfloat16),
        scratch_shapes=[
            pltpu.VMEM((2, CHUNK_M, CHUNK_COLS), jnp.bfloat16),
            pltpu.VMEM((2, 2, CHUNK_M, CHUNK_COLS), jnp.bfloat16),
            pltpu.VMEM((2, CHUNK_M, CHUNK_COLS), jnp.bfloat16),
            pltpu.VMEM((2, K, CHUNK_COLS), jnp.bfloat16),
            pltpu.SemaphoreType.DMA((24,)),
            pltpu.SemaphoreType.DMA((24,)),
            pltpu.SemaphoreType.DMA((4,)),
        ],
        compiler_params=pltpu.CompilerParams(
            collective_id=0, vmem_limit_bytes=60 * 1024 * 1024),
    )(x_bf, w_bf, scale_x, scale_w)
    return out.astype(jnp.float32)
